# baseline (speedup 1.0000x reference)
import jax
import jax.numpy as jnp
from jax.experimental import pallas as pl
from jax.experimental.pallas import tpu as pltpu

_B = 4096
_D = 32
_BLK = 256
_GRID = _B // _BLK
_K = 129
_TOP = 0.95
_BOT = 0.05
_INV_T = 1.0 / 0.07
_FILL_W = 512
_BISECT_ITERS = 18


def _norm_body(b_ref, bn_ref):
    b = b_ref[...]
    bn_ref[...] = b / jnp.maximum(
        jnp.sqrt(jnp.sum(b * b, axis=1, keepdims=True)), 1e-12)


def _body(a_ref, bn_ref, out_ref):
    pid = pl.program_id(0)
    a = a_ref[...]
    an = a / jnp.maximum(jnp.sqrt(jnp.sum(a * a, axis=1, keepdims=True)), 1e-12)
    bn = bn_ref[...]
    sim = jax.lax.dot_general(
        an, bn, (((1,), (1,)), ((), ())),
        preferred_element_type=jnp.float32)

    bnb = bn_ref[pl.ds(pid * _BLK, _BLK), :]
    pos = jnp.sum(an * bnb, axis=1, keepdims=True)

    col = jax.lax.broadcasted_iota(jnp.int32, sim.shape, 1)
    row = jax.lax.broadcasted_iota(jnp.int32, sim.shape, 0) + pid * _BLK
    diag = col == row
    m = (sim >= _BOT * pos) & (sim <= _TOP * pos) & jnp.logical_not(diag)
    vm = jnp.where(m, sim, -2.0)
    n = jnp.sum(m.astype(jnp.int32), axis=1, keepdims=True)
    nf = n.astype(jnp.float32)
    e = jnp.exp(sim * _INV_T)
    epos = jnp.exp(pos * _INV_T)

    lo = _BOT * pos
    hi = pos
    for _ in range(_BISECT_ITERS):
        mid = 0.5 * (lo + hi)
        c = jnp.sum((vm >= mid).astype(jnp.int32), axis=1, keepdims=True)
        ge = c >= _K
        lo = jnp.where(ge, mid, lo)
        hi = jnp.where(ge, hi, mid)
    t = lo
    gt = vm > t
    c_gt = jnp.sum(jnp.where(gt, 1.0, 0.0), axis=1, keepdims=True)
    s_a = (jnp.sum(jnp.where(gt, e, 0.0), axis=1, keepdims=True)
           + (_K - c_gt) * jnp.exp(t * _INV_T) + epos)

    nm = jnp.where(m[:, :_FILL_W], 0.0, 1.0)
    kk = jax.lax.broadcasted_iota(jnp.int32, (_FILL_W, _FILL_W), 0)
    ll = jax.lax.broadcasted_iota(jnp.int32, (_FILL_W, _FILL_W), 1)
    tri = jnp.where(kk <= ll, 1.0, 0.0)
    cum = jax.lax.dot_general(
        nm, tri, (((1,), (0,)), ((), ())),
        preferred_element_type=jnp.float32,
        precision=jax.lax.Precision.HIGHEST)
    fill = (nm > 0.0) & (cum <= (_K - nf))
    s_b = (jnp.sum(jnp.where(m | diag, e, 0.0), axis=1, keepdims=True)
           + jnp.sum(jnp.where(fill & jnp.logical_not(diag[:, :_FILL_W]),
                               e[:, :_FILL_W], 0.0), axis=1, keepdims=True))

    s = jnp.where(nf >= _K, s_a, s_b)
    loss = jnp.log(s) - pos * _INV_T
    block_total = jnp.sum(loss)

    lane = jax.lax.broadcasted_iota(jnp.int32, (1, 1, 128), 2)
    out_ref[...] = jnp.where(lane == 0, block_total, 0.0)


def kernel(embedding_A, embedding_B):
    bn = pl.pallas_call(
        _norm_body,
        out_shape=jax.ShapeDtypeStruct((_B, _D), jnp.float32),
    )(embedding_B)
    partial = pl.pallas_call(
        _body,
        grid=(_GRID,),
        in_specs=[
            pl.BlockSpec((_BLK, _D), lambda i: (i, 0)),
            pl.BlockSpec((_B, _D), lambda i: (0, 0)),
        ],
        out_specs=pl.BlockSpec((1, 1, 128), lambda i: (i, 0, 0)),
        out_shape=jax.ShapeDtypeStruct((_GRID, 1, 128), jnp.float32),
        compiler_params=pltpu.CompilerParams(
            dimension_semantics=("parallel",)),
    )(embedding_A, bn)
    return jnp.sum(partial[:, 0, 0]) / _B


# speedup vs baseline: 1.0966x; 1.0966x over previous
import jax
import jax.numpy as jnp
from jax.experimental import pallas as pl
from jax.experimental.pallas import tpu as pltpu

_B = 4096
_D = 32
_BLK = 256
_GRID = _B // _BLK
_K = 129
_TOP = 0.95
_BOT = 0.05
_INV_T = 1.0 / 0.07
_FILL_W = 512
_BISECT_ITERS = 16


def _body(a_ref, b_ref, out_ref, bn_ref):
    pid = pl.program_id(0)

    @pl.when(pid == 0)
    def _():
        b = b_ref[...]
        bn_ref[...] = b / jnp.maximum(
            jnp.sqrt(jnp.sum(b * b, axis=1, keepdims=True)), 1e-12)

    a = a_ref[...]
    an = a / jnp.maximum(jnp.sqrt(jnp.sum(a * a, axis=1, keepdims=True)), 1e-12)
    bn = bn_ref[...]
    sim = jax.lax.dot_general(
        an, bn, (((1,), (1,)), ((), ())),
        preferred_element_type=jnp.float32)

    bnb = bn_ref[pl.ds(pid * _BLK, _BLK), :]
    pos = jnp.sum(an * bnb, axis=1, keepdims=True)

    col = jax.lax.broadcasted_iota(jnp.int32, sim.shape, 1)
    row = jax.lax.broadcasted_iota(jnp.int32, sim.shape, 0) + pid * _BLK
    diag = col == row
    m = (sim >= _BOT * pos) & (sim <= _TOP * pos) & jnp.logical_not(diag)
    vm = jnp.where(m, sim, -2.0)
    n = jnp.sum(m.astype(jnp.int32), axis=1, keepdims=True)
    nf = n.astype(jnp.float32)
    e = jnp.exp(sim * _INV_T)
    epos = jnp.exp(pos * _INV_T)

    lo = _BOT * pos
    hi = pos
    for _ in range(_BISECT_ITERS):
        mid = 0.5 * (lo + hi)
        c = jnp.sum((vm >= mid).astype(jnp.int32), axis=1, keepdims=True)
        ge = c >= _K
        lo = jnp.where(ge, mid, lo)
        hi = jnp.where(ge, hi, mid)
    t = lo
    gt = vm > t
    c_gt = jnp.sum(gt.astype(jnp.int32), axis=1, keepdims=True).astype(jnp.float32)
    s_a = (jnp.sum(jnp.where(gt, e, 0.0), axis=1, keepdims=True)
           + (_K - c_gt) * jnp.exp(t * _INV_T) + epos)

    nm = jnp.where(m[:, :_FILL_W], 0.0, 1.0)
    kk = jax.lax.broadcasted_iota(jnp.int32, (_FILL_W, _FILL_W), 0)
    ll = jax.lax.broadcasted_iota(jnp.int32, (_FILL_W, _FILL_W), 1)
    tri = jnp.where(kk <= ll, 1.0, 0.0)
    cum = jax.lax.dot_general(
        nm, tri, (((1,), (0,)), ((), ())),
        preferred_element_type=jnp.float32,
        precision=jax.lax.Precision.HIGHEST)
    fill = (nm > 0.0) & (cum <= (_K - nf))
    s_b = (jnp.sum(jnp.where(m | diag, e, 0.0), axis=1, keepdims=True)
           + jnp.sum(jnp.where(fill & jnp.logical_not(diag[:, :_FILL_W]),
                               e[:, :_FILL_W], 0.0), axis=1, keepdims=True))

    s = jnp.where(nf >= _K, s_a, s_b)
    loss = jnp.log(s) - pos * _INV_T
    block_total = jnp.sum(loss)

    @pl.when(pid == 0)
    def _():
        out_ref[...] = jnp.zeros_like(out_ref)

    out_ref[...] += block_total.reshape(1, 1)


def kernel(embedding_A, embedding_B):
    total = pl.pallas_call(
        _body,
        grid=(_GRID,),
        in_specs=[
            pl.BlockSpec((_BLK, _D), lambda i: (i, 0)),
            pl.BlockSpec((_B, _D), lambda i: (0, 0)),
        ],
        out_specs=pl.BlockSpec((1, 1), lambda i: (0, 0)),
        out_shape=jax.ShapeDtypeStruct((1, 1), jnp.float32),
        scratch_shapes=[pltpu.VMEM((_B, _D), jnp.float32)],
    )(embedding_A, embedding_B)
    return total[0, 0] / _B


# speedup vs baseline: 1.2830x; 1.1700x over previous
import jax
import jax.numpy as jnp
from jax.experimental import pallas as pl
from jax.experimental.pallas import tpu as pltpu

_B = 4096
_D = 32
_BLK = 256
_GRID = _B // _BLK
_K = 129
_TOP = 0.95
_BOT = 0.05
_INV_T = 1.0 / 0.07
_FILL_W = 512
_BISECT_ITERS = 14


def _body(a_ref, b_ref, out_ref, bn_ref):
    pid = pl.program_id(0)

    @pl.when(pid == 0)
    def _():
        b = b_ref[...]
        bn_ref[...] = b / jnp.maximum(
            jnp.sqrt(jnp.sum(b * b, axis=1, keepdims=True)), 1e-12)

    a = a_ref[...]
    an = a / jnp.maximum(jnp.sqrt(jnp.sum(a * a, axis=1, keepdims=True)), 1e-12)
    bn = bn_ref[...]
    sim = jax.lax.dot_general(
        an, bn, (((1,), (1,)), ((), ())),
        preferred_element_type=jnp.float32)

    bnb = bn_ref[pl.ds(pid * _BLK, _BLK), :]
    pos = jnp.sum(an * bnb, axis=1, keepdims=True)

    m = (sim >= _BOT * pos) & (sim <= _TOP * pos)
    vm = jnp.where(m, sim, -2.0)
    n = jnp.sum(m.astype(jnp.int32), axis=1, keepdims=True)
    nf = n.astype(jnp.float32)
    e = jnp.exp(sim * _INV_T)
    epos = jnp.exp(pos * _INV_T)

    lo = _BOT * pos
    hi = pos
    for _ in range(_BISECT_ITERS):
        mid = 0.5 * (lo + hi)
        c = jnp.sum((vm >= mid).astype(jnp.int32), axis=1, keepdims=True)
        ge = c >= _K
        lo = jnp.where(ge, mid, lo)
        hi = jnp.where(ge, hi, mid)
    t = lo
    gt = vm > t
    c_gt = jnp.sum(gt.astype(jnp.int32), axis=1, keepdims=True).astype(jnp.float32)
    s_a = (jnp.sum(jnp.where(gt, e, 0.0), axis=1, keepdims=True)
           + (_K - c_gt) * jnp.exp(t * _INV_T) + epos)

    nm = jnp.where(m[:, :_FILL_W], 0.0, 1.0)
    kk = jax.lax.broadcasted_iota(jnp.int32, (_FILL_W, _FILL_W), 0)
    ll = jax.lax.broadcasted_iota(jnp.int32, (_FILL_W, _FILL_W), 1)
    tri = jnp.where(kk <= ll, 1.0, 0.0)
    cum = jax.lax.dot_general(
        nm, tri, (((1,), (0,)), ((), ())),
        preferred_element_type=jnp.float32)
    fill = (nm > 0.0) & (cum <= (_K - nf))
    col_w = jax.lax.broadcasted_iota(jnp.int32, (_BLK, _FILL_W), 1)
    row_w = jax.lax.broadcasted_iota(jnp.int32, (_BLK, _FILL_W), 0) + pid * _BLK
    s_b = (jnp.sum(jnp.where(m, e, 0.0), axis=1, keepdims=True) + epos
           + jnp.sum(jnp.where(fill & (col_w != row_w),
                               e[:, :_FILL_W], 0.0), axis=1, keepdims=True))

    s = jnp.where(nf >= _K, s_a, s_b)
    loss = jnp.log(s) - pos * _INV_T
    block_total = jnp.sum(loss)

    @pl.when(pid == 0)
    def _():
        out_ref[...] = jnp.zeros_like(out_ref)

    out_ref[...] += block_total.reshape(1, 1)


def kernel(embedding_A, embedding_B):
    total = pl.pallas_call(
        _body,
        grid=(_GRID,),
        in_specs=[
            pl.BlockSpec((_BLK, _D), lambda i: (i, 0)),
            pl.BlockSpec((_B, _D), lambda i: (0, 0)),
        ],
        out_specs=pl.BlockSpec((1, 1), lambda i: (0, 0)),
        out_shape=jax.ShapeDtypeStruct((1, 1), jnp.float32),
        scratch_shapes=[pltpu.VMEM((_B, _D), jnp.float32)],
    )(embedding_A, embedding_B)
    return total[0, 0] / _B


# speedup vs baseline: 1.3904x; 1.0837x over previous
import jax
import jax.numpy as jnp
from jax.experimental import pallas as pl
from jax.experimental.pallas import tpu as pltpu

_B = 4096
_D = 32
_BLK = 256
_GRID = _B // _BLK
_K = 129
_TOP = 0.95
_BOT = 0.05
_INV_T = 1.0 / 0.07
_INV_T_LOG2E = 1.4426950408889634 / 0.07
_FILL_W = 512
_BISECT_ITERS = 13


def _body(a_ref, b_ref, out_ref, bn_ref):
    pid = pl.program_id(0)

    @pl.when(pid == 0)
    def _():
        b = b_ref[...]
        bn_ref[...] = b / jnp.maximum(
            jnp.sqrt(jnp.sum(b * b, axis=1, keepdims=True)), 1e-12)

    a = a_ref[...]
    an = a / jnp.maximum(jnp.sqrt(jnp.sum(a * a, axis=1, keepdims=True)), 1e-12)
    bn = bn_ref[...]
    sim = jax.lax.dot_general(
        an, bn, (((1,), (1,)), ((), ())),
        preferred_element_type=jnp.float32)

    bnb = bn_ref[pl.ds(pid * _BLK, _BLK), :]
    pos = jnp.sum(an * bnb, axis=1, keepdims=True)

    m = (sim >= _BOT * pos) & (sim <= _TOP * pos)
    vm = jnp.where(m, sim, -2.0)
    e = jnp.exp2(sim * _INV_T_LOG2E)
    epos = jnp.exp2(pos * _INV_T_LOG2E)

    lo = _BOT * pos
    hi = pos
    for _ in range(_BISECT_ITERS):
        mid = 0.5 * (lo + hi)
        c = jnp.sum((vm >= mid).astype(jnp.int32), axis=1, keepdims=True)
        ge = c >= _K
        lo = jnp.where(ge, mid, lo)
        hi = jnp.where(ge, hi, mid)
    t = lo
    ge = vm >= t
    c_ge = jnp.sum(jnp.where(ge, 1.0, 0.0), axis=1, keepdims=True)
    nf = c_ge
    s_a = (jnp.sum(jnp.where(ge, e, 0.0), axis=1, keepdims=True)
           + (_K - c_ge) * jnp.exp2(t * _INV_T_LOG2E) + epos)

    nm = jnp.where(m[:, :_FILL_W], 0.0, 1.0)
    kk = jax.lax.broadcasted_iota(jnp.int32, (_FILL_W, _FILL_W), 0)
    ll = jax.lax.broadcasted_iota(jnp.int32, (_FILL_W, _FILL_W), 1)
    tri = jnp.where(kk <= ll, 1.0, 0.0)
    cum = jax.lax.dot_general(
        nm, tri, (((1,), (0,)), ((), ())),
        preferred_element_type=jnp.float32)
    fill = (nm > 0.0) & (cum <= (_K - nf))
    col_w = jax.lax.broadcasted_iota(jnp.int32, (_BLK, _FILL_W), 1)
    row_w = jax.lax.broadcasted_iota(jnp.int32, (_BLK, _FILL_W), 0) + pid * _BLK
    s_b = (jnp.sum(jnp.where(m, e, 0.0), axis=1, keepdims=True) + epos
           + jnp.sum(jnp.where(fill & (col_w != row_w),
                               e[:, :_FILL_W], 0.0), axis=1, keepdims=True))

    s = jnp.where(nf >= _K, s_a, s_b)
    loss = jnp.log(s) - pos * _INV_T
    block_total = jnp.sum(loss)

    @pl.when(pid == 0)
    def _():
        out_ref[...] = jnp.zeros_like(out_ref)

    out_ref[...] += block_total.reshape(1, 1)


def kernel(embedding_A, embedding_B):
    total = pl.pallas_call(
        _body,
        grid=(_GRID,),
        in_specs=[
            pl.BlockSpec((_BLK, _D), lambda i: (i, 0)),
            pl.BlockSpec((_B, _D), lambda i: (0, 0)),
        ],
        out_specs=pl.BlockSpec((1, 1), lambda i: (0, 0)),
        out_shape=jax.ShapeDtypeStruct((1, 1), jnp.float32),
        scratch_shapes=[pltpu.VMEM((_B, _D), jnp.float32)],
    )(embedding_A, embedding_B)
    return total[0, 0] / _B


# speedup vs baseline: 1.4362x; 1.0330x over previous
import jax
import jax.numpy as jnp
from jax.experimental import pallas as pl
from jax.experimental.pallas import tpu as pltpu

_B = 4096
_D = 32
_BLK = 512
_GRID = _B // _BLK
_K = 129
_TOP = 0.95
_BOT = 0.05
_INV_T = 1.0 / 0.07
_INV_T_LOG2E = 1.4426950408889634 / 0.07
_FILL_W = 512
_BISECT_ITERS = 13


def _body(a_ref, b_ref, out_ref, bn_ref):
    pid = pl.program_id(0)

    @pl.when(pid == 0)
    def _():
        b = b_ref[...]
        bn_ref[...] = b / jnp.maximum(
            jnp.sqrt(jnp.sum(b * b, axis=1, keepdims=True)), 1e-12)

    a = a_ref[...]
    an = a / jnp.maximum(jnp.sqrt(jnp.sum(a * a, axis=1, keepdims=True)), 1e-12)
    bn = bn_ref[...]
    sim = jax.lax.dot_general(
        an, bn, (((1,), (1,)), ((), ())),
        preferred_element_type=jnp.float32)

    bnb = bn_ref[pl.ds(pid * _BLK, _BLK), :]
    pos = jnp.sum(an * bnb, axis=1, keepdims=True)

    m = (sim >= _BOT * pos) & (sim <= _TOP * pos)
    vm = jnp.where(m, sim, -2.0)
    e = jnp.exp2(sim * _INV_T_LOG2E)
    epos = jnp.exp2(pos * _INV_T_LOG2E)

    lo = _BOT * pos
    hi = pos
    for _ in range(_BISECT_ITERS):
        mid = 0.5 * (lo + hi)
        c = jnp.sum((vm >= mid).astype(jnp.int32), axis=1, keepdims=True)
        ge = c >= _K
        lo = jnp.where(ge, mid, lo)
        hi = jnp.where(ge, hi, mid)
    t = lo
    ge = vm >= t
    c_ge = jnp.sum(jnp.where(ge, 1.0, 0.0), axis=1, keepdims=True)
    nf = c_ge
    s_a = (jnp.sum(jnp.where(ge, e, 0.0), axis=1, keepdims=True)
           + (_K - c_ge) * jnp.exp2(t * _INV_T_LOG2E) + epos)

    nm = jnp.where(m[:, :_FILL_W], 0.0, 1.0)
    kk = jax.lax.broadcasted_iota(jnp.int32, (_FILL_W, _FILL_W), 0)
    ll = jax.lax.broadcasted_iota(jnp.int32, (_FILL_W, _FILL_W), 1)
    tri = jnp.where(kk <= ll, 1.0, 0.0)
    cum = jax.lax.dot_general(
        nm, tri, (((1,), (0,)), ((), ())),
        preferred_element_type=jnp.float32)
    fill = (nm > 0.0) & (cum <= (_K - nf))
    col_w = jax.lax.broadcasted_iota(jnp.int32, (_BLK, _FILL_W), 1)
    row_w = jax.lax.broadcasted_iota(jnp.int32, (_BLK, _FILL_W), 0) + pid * _BLK
    s_b = (jnp.sum(jnp.where(m, e, 0.0), axis=1, keepdims=True) + epos
           + jnp.sum(jnp.where(fill & (col_w != row_w),
                               e[:, :_FILL_W], 0.0), axis=1, keepdims=True))

    s = jnp.where(nf >= _K, s_a, s_b)
    loss = jnp.log(s) - pos * _INV_T
    block_total = jnp.sum(loss)

    @pl.when(pid == 0)
    def _():
        out_ref[...] = jnp.zeros_like(out_ref)

    out_ref[...] += block_total.reshape(1, 1)


def kernel(embedding_A, embedding_B):
    total = pl.pallas_call(
        _body,
        grid=(_GRID,),
        in_specs=[
            pl.BlockSpec((_BLK, _D), lambda i: (i, 0)),
            pl.BlockSpec((_B, _D), lambda i: (0, 0)),
        ],
        out_specs=pl.BlockSpec((1, 1), lambda i: (0, 0)),
        out_shape=jax.ShapeDtypeStruct((1, 1), jnp.float32),
        scratch_shapes=[pltpu.VMEM((_B, _D), jnp.float32)],
    )(embedding_A, embedding_B)
    return total[0, 0] / _B
